# baseline (speedup 1.0000x reference)
import functools

import jax
import jax.numpy as jnp
from jax.experimental import pallas as pl

N = 10000
E = 320000
H = 128
NH = 2
ALPHA = 0.2


def _score_block(xs_ref, xd_ref, eh_ref, p1_ref, p2_ref, p3_ref, bp1_ref, wp2_ref, bp2_ref, out_ref):
    xs = xs_ref[...]
    xd = xd_ref[...]
    eh = eh_ref[...]
    h = (xs @ p1_ref[...] + xd @ p2_ref[...] + eh @ p3_ref[...]) + bp1_ref[...]
    h = jnp.maximum(h, 0.0)
    out_ref[...] = h @ wp2_ref[...] + bp2_ref[...]


def _score_pallas(xs, xd, eh, Wp1, bp1, Wp2, bp2):
    BE = 2000
    grid = (E // BE,)
    return pl.pallas_call(
        _score_block,
        grid=grid,
        in_specs=[
            pl.BlockSpec((BE, H), lambda i: (i, 0)),
            pl.BlockSpec((BE, H), lambda i: (i, 0)),
            pl.BlockSpec((BE, H), lambda i: (i, 0)),
            pl.BlockSpec((H, 64), lambda i: (0, 0)),
            pl.BlockSpec((H, 64), lambda i: (0, 0)),
            pl.BlockSpec((H, 64), lambda i: (0, 0)),
            pl.BlockSpec((64,), lambda i: (0,)),
            pl.BlockSpec((64, 1), lambda i: (0, 0)),
            pl.BlockSpec((1,), lambda i: (0,)),
        ],
        out_specs=pl.BlockSpec((BE, 1), lambda i: (i, 0)),
        out_shape=jax.ShapeDtypeStruct((E, 1), jnp.float32),
    )(xs, xd, eh, Wp1[:H], Wp1[H:2 * H], Wp1[2 * H:], bp1, Wp2, bp2)


def _gat_head(xh, eh, src, dst, Wn, bn, We, be, a):
    z = xh @ Wn + bn
    ze = eh @ We + be
    zs = jnp.take(z, src, axis=0)
    zd = jnp.take(z, dst, axis=0)
    a1, a2, a3 = a[:H], a[H:2 * H], a[2 * H:]
    logits = jax.nn.leaky_relu(zs @ a1 + zd @ a2 + ze @ a3, negative_slope=ALPHA)
    m = jax.ops.segment_max(logits, dst, num_segments=N)
    m = jnp.where(jnp.isfinite(m), m, 0.0)
    ex = jnp.exp(logits - jnp.take(m, dst, axis=0))
    s = jax.ops.segment_sum(ex, dst, num_segments=N)
    attn = ex / (jnp.take(s, dst, axis=0) + 1e-16)
    node_out = jax.nn.elu(jax.ops.segment_sum(attn[:, None] * zs, dst, num_segments=N))
    edge_out = jax.nn.elu(zs + zd + ze)
    return node_out, edge_out


def kernel(g, x, e, pe, adj_torch, W1n, b1n, W1e, b1e, W2n, b2n, W2e, b2e, gat_Wn, gat_We, gat_bn, gat_be, gat_a, Wp1, bp1, Wp2, bp2):
    src = g[0]
    dst = g[1]
    degree = pe[:, 0:2]
    xh = degree @ W1n + b1n
    eh = e @ W1e + b1e
    for l in range(2):
        n_outs = []
        e_outs = []
        for h in range(NH):
            no, eo = _gat_head(xh, eh, src, dst, gat_Wn[l, h], gat_bn[l, h], gat_We[l, h], gat_be[l, h], gat_a[l, h])
            n_outs.append(no)
            e_outs.append(eo)
        xh = jnp.concatenate(n_outs, axis=-1) @ W2n + b2n
        eh = jnp.concatenate(e_outs, axis=-1) @ W2e + b2e
    xs = jnp.take(xh, src, axis=0)
    xd = jnp.take(xh, dst, axis=0)
    return _score_pallas(xs, xd, eh, Wp1, bp1, Wp2, bp2)


# speedup vs baseline: 1.0874x; 1.0874x over previous
import functools

import jax
import jax.numpy as jnp
from jax.experimental import pallas as pl

N = 10000
E = 320000
H = 128
NH = 2
HES = 64
ALPHA = 0.2
BE = 2000


def _elu(x):
    return jnp.where(x > 0, x, jnp.exp(x) - 1.0)


def _edge_layer_block(ef_ref, g0_ref, g1_ref, ls_ref,
                      w0_ref, b0_ref, w1_ref, b1_ref,
                      a30_ref, a31_ref, w2a_ref, w2b_ref, b2_ref,
                      eh_out_ref, ex_out_ref):
    ef = ef_ref[...]
    ze0 = ef @ w0_ref[...] + b0_ref[...]
    ze1 = ef @ w1_ref[...] + b1_ref[...]
    l0 = ls_ref[:, 0:1] + ze0 @ a30_ref[...]
    l1 = ls_ref[:, 1:2] + ze1 @ a31_ref[...]
    l0 = jnp.where(l0 > 0, l0, ALPHA * l0)
    l1 = jnp.where(l1 > 0, l1, ALPHA * l1)
    ex_out_ref[:, 0:1] = jnp.exp(l0)
    ex_out_ref[:, 1:2] = jnp.exp(l1)
    eo0 = _elu(g0_ref[...] + ze0)
    eo1 = _elu(g1_ref[...] + ze1)
    eh_out_ref[...] = eo0 @ w2a_ref[...] + eo1 @ w2b_ref[...] + b2_ref[...]


def _edge_layer(ef, g0, g1, ls, w0, b0, w1, b1, a30, a31, w2a, w2b, b2):
    F = ef.shape[1]
    grid = (E // BE,)
    return pl.pallas_call(
        _edge_layer_block,
        grid=grid,
        in_specs=[
            pl.BlockSpec((BE, F), lambda i: (i, 0)),
            pl.BlockSpec((BE, H), lambda i: (i, 0)),
            pl.BlockSpec((BE, H), lambda i: (i, 0)),
            pl.BlockSpec((BE, 2), lambda i: (i, 0)),
            pl.BlockSpec((F, H), lambda i: (0, 0)),
            pl.BlockSpec((H,), lambda i: (0,)),
            pl.BlockSpec((F, H), lambda i: (0, 0)),
            pl.BlockSpec((H,), lambda i: (0,)),
            pl.BlockSpec((H, 1), lambda i: (0, 0)),
            pl.BlockSpec((H, 1), lambda i: (0, 0)),
            pl.BlockSpec((H, H), lambda i: (0, 0)),
            pl.BlockSpec((H, H), lambda i: (0, 0)),
            pl.BlockSpec((H,), lambda i: (0,)),
        ],
        out_specs=[
            pl.BlockSpec((BE, H), lambda i: (i, 0)),
            pl.BlockSpec((BE, 2), lambda i: (i, 0)),
        ],
        out_shape=[
            jax.ShapeDtypeStruct((E, H), jnp.float32),
            jax.ShapeDtypeStruct((E, 2), jnp.float32),
        ],
    )(ef, g0, g1, ls, w0, b0, w1, b1, a30, a31, w2a, w2b, b2)


def _score_block(r_ref, eh_ref, wp1c_ref, bp1_ref, wp2_ref, bp2_ref, out_ref):
    h = r_ref[...] + eh_ref[...] @ wp1c_ref[...] + bp1_ref[...]
    h = jnp.maximum(h, 0.0)
    out_ref[...] = h @ wp2_ref[...] + bp2_ref[...]


def _score_pallas(r, eh, Wp1c, bp1, Wp2, bp2):
    grid = (E // BE,)
    return pl.pallas_call(
        _score_block,
        grid=grid,
        in_specs=[
            pl.BlockSpec((BE, HES), lambda i: (i, 0)),
            pl.BlockSpec((BE, H), lambda i: (i, 0)),
            pl.BlockSpec((H, HES), lambda i: (0, 0)),
            pl.BlockSpec((HES,), lambda i: (0,)),
            pl.BlockSpec((HES, 1), lambda i: (0, 0)),
            pl.BlockSpec((1,), lambda i: (0,)),
        ],
        out_specs=pl.BlockSpec((BE, 1), lambda i: (i, 0)),
        out_shape=jax.ShapeDtypeStruct((E, 1), jnp.float32),
    )(r, eh, Wp1c, bp1, Wp2, bp2)


def kernel(g, x, e, pe, adj_torch, W1n, b1n, W1e, b1e, W2n, b2n, W2e, b2e,
           gat_Wn, gat_We, gat_bn, gat_be, gat_a, Wp1, bp1, Wp2, bp2):
    src = g[0]
    dst = g[1]
    xh = pe[:, 0:2] @ W1n + b1n
    ef = e
    for l in range(2):
        z = []
        ls_cols = []
        w_eff = []
        b_eff = []
        for h in range(NH):
            zh = xh @ gat_Wn[l, h] + gat_bn[l, h]
            a = gat_a[l, h]
            u = zh @ a[:H]
            v = zh @ a[H:2 * H]
            ls_cols.append(jnp.take(u, src) + jnp.take(v, dst))
            z.append(zh)
            if l == 0:
                w_eff.append(W1e @ gat_We[0, h])
                b_eff.append(b1e @ gat_We[0, h] + gat_be[0, h])
            else:
                w_eff.append(gat_We[l, h])
                b_eff.append(gat_be[l, h])
        zs0 = jnp.take(z[0], src, axis=0)
        zs1 = jnp.take(z[1], src, axis=0)
        g0 = zs0 + jnp.take(z[0], dst, axis=0)
        g1 = zs1 + jnp.take(z[1], dst, axis=0)
        ls = jnp.stack(ls_cols, axis=1)
        a30 = gat_a[l, 0][2 * H:][:, None]
        a31 = gat_a[l, 1][2 * H:][:, None]
        eh_next, ex = _edge_layer(ef, g0, g1, ls,
                                  w_eff[0], b_eff[0], w_eff[1], b_eff[1],
                                  a30, a31, W2e[:H], W2e[H:], b2e)
        num0 = jax.ops.segment_sum(ex[:, 0:1] * zs0, dst, num_segments=N)
        num1 = jax.ops.segment_sum(ex[:, 1:2] * zs1, dst, num_segments=N)
        s0 = jax.ops.segment_sum(ex[:, 0], dst, num_segments=N)
        s1 = jax.ops.segment_sum(ex[:, 1], dst, num_segments=N)
        n0 = jax.nn.elu(num0 / (s0[:, None] + 1e-16))
        n1 = jax.nn.elu(num1 / (s1[:, None] + 1e-16))
        xh = jnp.concatenate([n0, n1], axis=-1) @ W2n + b2n
        ef = eh_next
    p = xh @ Wp1[:H]
    q = xh @ Wp1[H:2 * H]
    r = jnp.take(p, src, axis=0) + jnp.take(q, dst, axis=0)
    return _score_pallas(r, ef, Wp1[2 * H:], bp1, Wp2, bp2)


# speedup vs baseline: 5.3279x; 4.8998x over previous
import functools

import jax
import jax.numpy as jnp
from jax import lax
from jax.experimental import pallas as pl
from jax.experimental.pallas import tpu as pltpu
from jax.experimental.pallas import tpu_sc as plsc

N = 10000
E = 320000
H = 128
NH = 2
HES = 64
ALPHA = 0.2
BE = 2000

TW = 144
NP = 10240
RPT = NP // 16
UCOL = 128
VCOL = 129
ONECOL = 130
CH = 80
EPT = E // 16
SW = HES
CH2 = 400
EPT2 = E // 32


def _elu(x):
    return jnp.where(x > 0, x, jnp.exp(x) - 1.0)


def _gat_sc_layer(src, dst, table, ehq):
    mesh = plsc.VectorSubcoreMesh(core_axis_name="c", subcore_axis_name="s")

    @functools.partial(
        pl.kernel, mesh=mesh,
        compiler_params=pltpu.CompilerParams(use_tc_tiling_on_sc=False, needs_layout_passes=False),
        out_type=[
            jax.ShapeDtypeStruct((2 * E, H), jnp.float32),
            jax.ShapeDtypeStruct((2 * NP, TW), jnp.float32),
        ],
        scratch_types=[
            pltpu.VMEM((CH,), jnp.int32),
            pltpu.VMEM((CH,), jnp.int32),
            pltpu.VMEM((CH,), jnp.int32),
            pltpu.VMEM((CH,), jnp.int32),
            pltpu.VMEM((CH,), jnp.float32),
            pltpu.VMEM((CH,), jnp.float32),
            pltpu.VMEM((CH, TW), jnp.float32),
            pltpu.VMEM((CH, TW), jnp.float32),
            pltpu.VMEM((CH, H), jnp.float32),
            pltpu.VMEM_SHARED((NP, TW), jnp.float32),
            pltpu.SemaphoreType.DMA,
            pltpu.SemaphoreType.DMA,
        ],
    )
    def k(src_hbm, dst_hbm, table_hbm, ehq_hbm, g_out, acc_out,
          src_v, dst_v, srcs_v, dsts_v, ehq_v, ex_v, A, B, G, accum,
          sem1, sem2):
        c = lax.axis_index("c")
        s = lax.axis_index("s")
        iota = lax.iota(jnp.int32, 16)

        zrow = jnp.zeros((16,), jnp.float32)
        def zero_body(i, _):
            for r in range(TW // 16):
                plsc.store_scatter(A, [jnp.full((16,), i, jnp.int32),
                                       iota + 16 * r], zrow)
            return 0
        lax.fori_loop(0, CH, zero_body, 0)
        for j in range(RPT // CH):
            pltpu.sync_copy(A, accum.at[pl.ds(s * RPT + j * CH, CH), :])
        plsc.subcore_barrier()

        cN = c * N

        def chunk_body(kk, _):
            base = s * EPT + kk * CH
            pltpu.sync_copy(src_hbm.at[pl.ds(base, CH)], src_v)
            pltpu.sync_copy(dst_hbm.at[pl.ds(base, CH)], dst_v)
            pltpu.sync_copy(ehq_hbm.at[pl.ds(c * E + base, CH)], ehq_v)
            for j in range(CH // 16):
                sl = pl.ds(16 * j, 16)
                srcs_v[sl] = src_v[sl] + cN
                dsts_v[sl] = dst_v[sl] + cN
            cpa = pltpu.async_copy(table_hbm.at[srcs_v], A, sem1)
            cpb = pltpu.async_copy(table_hbm.at[dsts_v], B, sem2)
            cpa.wait()
            cpb.wait()
            ucol = jnp.full((16,), UCOL, jnp.int32)
            vcol = jnp.full((16,), VCOL, jnp.int32)
            for j in range(CH // 16):
                sl = pl.ds(16 * j, 16)
                eids = iota + 16 * j
                au = plsc.load_gather(A, [eids, ucol])
                bv = plsc.load_gather(B, [eids, vcol])
                lg = au + bv + ehq_v[sl]
                lg = jnp.where(lg > 0, lg, ALPHA * lg)
                ex_v[sl] = jnp.exp(lg)

            def edge_body(e2, _):
                eful = jnp.full((16,), 0, jnp.int32) + e2
                exs = plsc.load_gather(ex_v, [eful])
                for r in range(TW // 16):
                    cols = iota + 16 * r
                    a = plsc.load_gather(A, [eful, cols])
                    if r < H // 16:
                        b = plsc.load_gather(B, [eful, cols])
                        plsc.store_scatter(G, [eful, cols], a + b)
                    plsc.store_scatter(A, [eful, cols], a * exs)
                return 0
            lax.fori_loop(0, CH, edge_body, 0)
            pltpu.sync_copy(G, g_out.at[pl.ds(c * E + base, CH), :])
            pltpu.sync_copy(A, accum.at[dst_v], add=True)
            return 0
        lax.fori_loop(0, EPT // CH, chunk_body, 0)
        plsc.subcore_barrier()
        for j in range(RPT // 160):
            pltpu.sync_copy(
                accum.at[pl.ds(s * RPT + j * 160, 160), :],
                acc_out.at[pl.ds(c * NP + s * RPT + j * 160, 160), :])

    return k(src, dst, table, ehq)


def _score_gather_sc(src, dst, table):
    mesh = plsc.VectorSubcoreMesh(core_axis_name="c", subcore_axis_name="s")

    @functools.partial(
        pl.kernel, mesh=mesh,
        compiler_params=pltpu.CompilerParams(use_tc_tiling_on_sc=False, needs_layout_passes=False),
        out_type=jax.ShapeDtypeStruct((E, SW), jnp.float32),
        scratch_types=[
            pltpu.VMEM((CH2,), jnp.int32),
            pltpu.VMEM((CH2,), jnp.int32),
            pltpu.VMEM((CH2, SW), jnp.float32),
            pltpu.VMEM((CH2, SW), jnp.float32),
            pltpu.SemaphoreType.DMA,
            pltpu.SemaphoreType.DMA,
        ],
    )
    def k(src_hbm, dst_hbm, table_hbm, r_out, src_v, dst_v, RA, RB,
          sem1, sem2):
        c = lax.axis_index("c")
        s = lax.axis_index("s")
        wid = s * 2 + c
        iota = lax.iota(jnp.int32, 16)

        def chunk_body(kk, _):
            base = wid * EPT2 + kk * CH2
            pltpu.sync_copy(src_hbm.at[pl.ds(base, CH2)], src_v)
            pltpu.sync_copy(dst_hbm.at[pl.ds(base, CH2)], dst_v)
            for j in range(CH2 // 16):
                sl = pl.ds(16 * j, 16)
                dst_v[sl] = dst_v[sl] + N
            cpa = pltpu.async_copy(table_hbm.at[src_v], RA, sem1)
            cpb = pltpu.async_copy(table_hbm.at[dst_v], RB, sem2)
            cpa.wait()
            cpb.wait()

            def add_body(e2, _):
                eful = jnp.full((16,), 0, jnp.int32) + e2
                for r in range(SW // 16):
                    cols = iota + 16 * r
                    a = plsc.load_gather(RA, [eful, cols])
                    b = plsc.load_gather(RB, [eful, cols])
                    plsc.store_scatter(RA, [eful, cols], a + b)
                return 0
            lax.fori_loop(0, CH2, add_body, 0)
            pltpu.sync_copy(RA, r_out.at[pl.ds(base, CH2), :])
            return 0
        lax.fori_loop(0, EPT2 // CH2, chunk_body, 0)

    return k(src, dst, table)


def _edge_layer_block(ef_ref, g0_ref, g1_ref,
                      w0_ref, b0_ref, w1_ref, b1_ref,
                      w2a_ref, w2b_ref, b2_ref, qn_ref, cn_ref,
                      eh_out_ref, ehq_out_ref):
    ef = ef_ref[...]
    ze0 = ef @ w0_ref[...] + b0_ref[...]
    ze1 = ef @ w1_ref[...] + b1_ref[...]
    eo0 = _elu(g0_ref[...] + ze0)
    eo1 = _elu(g1_ref[...] + ze1)
    eh = eo0 @ w2a_ref[...] + eo1 @ w2b_ref[...] + b2_ref[...]
    eh_out_ref[...] = eh
    ehq_out_ref[...] = eh @ qn_ref[...] + cn_ref[...]


def _edge_layer(ef, g0, g1, w0, b0, w1, b1, w2a, w2b, b2, qn, cn):
    F = ef.shape[1]
    grid = (E // BE,)
    return pl.pallas_call(
        _edge_layer_block,
        grid=grid,
        in_specs=[
            pl.BlockSpec((BE, F), lambda i: (i, 0)),
            pl.BlockSpec((BE, H), lambda i: (i, 0)),
            pl.BlockSpec((BE, H), lambda i: (i, 0)),
            pl.BlockSpec((F, H), lambda i: (0, 0)),
            pl.BlockSpec((H,), lambda i: (0,)),
            pl.BlockSpec((F, H), lambda i: (0, 0)),
            pl.BlockSpec((H,), lambda i: (0,)),
            pl.BlockSpec((H, H), lambda i: (0, 0)),
            pl.BlockSpec((H, H), lambda i: (0, 0)),
            pl.BlockSpec((H,), lambda i: (0,)),
            pl.BlockSpec((H, 2), lambda i: (0, 0)),
            pl.BlockSpec((2,), lambda i: (0,)),
        ],
        out_specs=[
            pl.BlockSpec((BE, H), lambda i: (i, 0)),
            pl.BlockSpec((BE, 2), lambda i: (i, 0)),
        ],
        out_shape=[
            jax.ShapeDtypeStruct((E, H), jnp.float32),
            jax.ShapeDtypeStruct((E, 2), jnp.float32),
        ],
    )(ef, g0, g1, w0, b0, w1, b1, w2a, w2b, b2, qn, cn)


def _score_block(r_ref, eh_ref, wp1c_ref, bp1_ref, wp2_ref, bp2_ref, out_ref):
    h = r_ref[...] + eh_ref[...] @ wp1c_ref[...] + bp1_ref[...]
    h = jnp.maximum(h, 0.0)
    out_ref[...] = h @ wp2_ref[...] + bp2_ref[...]


def _score_pallas(r, eh, Wp1c, bp1, Wp2, bp2):
    grid = (E // BE,)
    return pl.pallas_call(
        _score_block,
        grid=grid,
        in_specs=[
            pl.BlockSpec((BE, HES), lambda i: (i, 0)),
            pl.BlockSpec((BE, H), lambda i: (i, 0)),
            pl.BlockSpec((H, HES), lambda i: (0, 0)),
            pl.BlockSpec((HES,), lambda i: (0,)),
            pl.BlockSpec((HES, 1), lambda i: (0, 0)),
            pl.BlockSpec((1,), lambda i: (0,)),
        ],
        out_specs=pl.BlockSpec((BE, 1), lambda i: (i, 0)),
        out_shape=jax.ShapeDtypeStruct((E, 1), jnp.float32),
    )(r, eh, Wp1c, bp1, Wp2, bp2)


def kernel(g, x, e, pe, adj_torch, W1n, b1n, W1e, b1e, W2n, b2n, W2e, b2e,
           gat_Wn, gat_We, gat_bn, gat_be, gat_a, Wp1, bp1, Wp2, bp2):
    src = g[0]
    dst = g[1]
    xh = pe[:, 0:2] @ W1n + b1n
    ef = e

    w_eff = [[W1e @ gat_We[0, 0], W1e @ gat_We[0, 1]],
             [gat_We[1, 0], gat_We[1, 1]]]
    b_eff = [[b1e @ gat_We[0, 0] + gat_be[0, 0],
              b1e @ gat_We[0, 1] + gat_be[0, 1]],
             [gat_be[1, 0], gat_be[1, 1]]]
    a3 = [[gat_a[l, h][2 * H:] for h in range(NH)] for l in range(2)]

    q0 = jnp.stack([w_eff[0][0] @ a3[0][0], w_eff[0][1] @ a3[0][1]], axis=1)
    c0 = jnp.stack([b_eff[0][0] @ a3[0][0], b_eff[0][1] @ a3[0][1]])
    ehq = (e @ q0 + c0)
    ehq = ehq.T.reshape(2 * E)

    ones = jnp.ones((N, 1), jnp.float32)
    padz = jnp.zeros((N, TW - ONECOL - 1), jnp.float32)

    for l in range(2):
        tabs = []
        for h in range(NH):
            zh = xh @ gat_Wn[l, h] + gat_bn[l, h]
            a = gat_a[l, h]
            u = (zh @ a[:H])[:, None]
            v = (zh @ a[H:2 * H])[:, None]
            tabs.append(jnp.concatenate([zh, u, v, ones, padz], axis=1))
        table = jnp.concatenate(tabs, axis=0)

        gout, acc = _gat_sc_layer(src, dst, table, ehq)

        if l == 0:
            qn = jnp.stack([gat_We[1, 0] @ a3[1][0],
                            gat_We[1, 1] @ a3[1][1]], axis=1)
            cn = jnp.stack([gat_be[1, 0] @ a3[1][0],
                            gat_be[1, 1] @ a3[1][1]])
        else:
            qn = jnp.zeros((H, 2), jnp.float32)
            cn = jnp.zeros((2,), jnp.float32)
        eh_next, ehq_next = _edge_layer(
            ef, gout[:E], gout[E:],
            w_eff[l][0], b_eff[l][0], w_eff[l][1], b_eff[l][1],
            W2e[:H], W2e[H:], b2e, qn, cn)

        n_parts = []
        for h in range(NH):
            num = acc[h * NP:h * NP + N, :H]
            ssum = acc[h * NP:h * NP + N, ONECOL]
            n_parts.append(jax.nn.elu(num / (ssum[:, None] + 1e-16)))
        xh = jnp.concatenate(n_parts, axis=-1) @ W2n + b2n
        ef = eh_next
        ehq = ehq_next.T.reshape(2 * E)

    p = xh @ Wp1[:H]
    q = xh @ Wp1[H:2 * H]
    r = _score_gather_sc(src, dst, jnp.concatenate([p, q], axis=0))
    return _score_pallas(r, ef, Wp1[2 * H:], bp1, Wp2, bp2)
